# baseline (speedup 1.0000x reference)
import functools

import jax
import jax.numpy as jnp
from jax import lax
from jax.experimental import pallas as pl
from jax.experimental.pallas import tpu as pltpu
from jax.experimental.pallas import tpu_sc as plsc

D_MODEL = 64
SCALE = 8.0

NUM_CORES = 2
NUM_SUBCORES = 16
NUM_WORKERS = NUM_CORES * NUM_SUBCORES

IDX_MINOR = 128
K = 8


def _emb_body(x_hbm, table_hbm, out_hbm, idx_v, rows_v, sem, *, chunks):
    wid = lax.axis_index("s") * NUM_CORES + lax.axis_index("c")
    row0 = wid * (chunks * K)

    def chunk(g, carry):
        r0 = row0 + g * K
        pltpu.sync_copy(x_hbm.at[pl.ds(r0, K)], idx_v)
        copies = [
            pltpu.async_copy(table_hbm.at[idx_v.at[j]], rows_v.at[j], sem)
            for j in range(K)
        ]
        for c in copies:
            c.wait()

        def scale(i, carry2):
            j = i // IDX_MINOR
            r = i % IDX_MINOR
            for c in range(D_MODEL // 16):
                sl = (j, r, pl.ds(c * 16, 16))
                rows_v[sl] = rows_v[sl] * SCALE
            return carry2

        lax.fori_loop(0, K * IDX_MINOR, scale, 0)
        pltpu.sync_copy(rows_v, out_hbm.at[pl.ds(r0, K)])
        return carry

    lax.fori_loop(0, chunks, chunk, 0)


@functools.partial(jax.jit, static_argnames=("chunks",))
def _emb(x2d, table, chunks):
    nrows = x2d.shape[0]
    mesh = plsc.VectorSubcoreMesh(core_axis_name="c", subcore_axis_name="s")
    kern = pl.kernel(
        functools.partial(_emb_body, chunks=chunks),
        out_type=jax.ShapeDtypeStruct((nrows, IDX_MINOR, D_MODEL), jnp.float32),
        mesh=mesh,
        scratch_types=[
            pltpu.VMEM((K, IDX_MINOR), jnp.int32),
            pltpu.VMEM((K, IDX_MINOR, D_MODEL), jnp.float32),
            pltpu.SemaphoreType.DMA,
        ],
        compiler_params=pltpu.CompilerParams(use_tc_tiling_on_sc=False),
    )
    return kern(x2d, table)


def kernel(x, table):
    b = x.size
    assert b % (NUM_WORKERS * K * IDX_MINOR) == 0
    chunks = b // (NUM_WORKERS * K * IDX_MINOR)
    x2d = x.reshape(b // IDX_MINOR, IDX_MINOR).astype(jnp.int32)
    out = _emb(x2d, table, chunks)
    return out.reshape(x.shape + (D_MODEL,))


# speedup vs baseline: 1.0949x; 1.0949x over previous
import functools

import jax
import jax.numpy as jnp
from jax import lax
from jax.experimental import pallas as pl
from jax.experimental.pallas import tpu as pltpu
from jax.experimental.pallas import tpu_sc as plsc

D_MODEL = 64
SCALE = 8.0

NUM_CORES = 2
NUM_SUBCORES = 16
NUM_WORKERS = NUM_CORES * NUM_SUBCORES

IDX_MINOR = 128
K = 5


def _emb_body(x_hbm, table_hbm, out_hbm,
              idx0, idx1, rows0, rows1, gsem0, gsem1, ssem0, ssem1,
              *, chunks):
    idx = (idx0, idx1)
    rows = (rows0, rows1)
    gsem = (gsem0, gsem1)
    ssem = (ssem0, ssem1)

    wid = lax.axis_index("s") * NUM_CORES + lax.axis_index("c")
    row0 = wid * (chunks * K)

    def fire(b, g):
        r0 = row0 + g * K
        pltpu.sync_copy(x_hbm.at[pl.ds(r0, K)], idx[b])
        for j in range(K):
            pltpu.async_copy(table_hbm.at[idx[b].at[j]], rows[b].at[j], gsem[b])

    def wait_gather(b):
        for j in range(K):
            pltpu.make_async_copy(
                table_hbm.at[idx[b].at[j]], rows[b].at[j], gsem[b]).wait()

    def scale(b):
        @plsc.parallel_loop(0, K * IDX_MINOR, unroll=4)
        def _(i):
            j = i // IDX_MINOR
            r = lax.rem(i, IDX_MINOR)
            for c in range(D_MODEL // 16):
                sl = (j, r, pl.ds(c * 16, 16))
                rows[b][sl] = rows[b][sl] * SCALE

    def store(b, g):
        r0 = row0 + g * K
        pltpu.async_copy(rows[b], out_hbm.at[pl.ds(r0, K)], ssem[b])

    def wait_store(b):
        pltpu.make_async_copy(rows[b], out_hbm.at[pl.ds(row0, K)], ssem[b]).wait()

    pairs = chunks // 2
    fire(0, 0)

    def pair(p, carry):
        g0 = 2 * p

        @pl.when(p > 0)
        def _():
            wait_store(1)

        fire(1, g0 + 1)
        wait_gather(0)
        scale(0)
        store(0, g0)
        wait_store(0)

        @pl.when(p < pairs - 1)
        def _():
            fire(0, g0 + 2)

        wait_gather(1)
        scale(1)
        store(1, g0 + 1)
        return carry

    lax.fori_loop(0, pairs, pair, 0)
    wait_store(1)


@functools.partial(jax.jit, static_argnames=("chunks",))
def _emb(x2d, table, chunks):
    nrows = x2d.shape[0]
    mesh = plsc.VectorSubcoreMesh(core_axis_name="c", subcore_axis_name="s")
    kern = pl.kernel(
        functools.partial(_emb_body, chunks=chunks),
        out_type=jax.ShapeDtypeStruct((nrows, IDX_MINOR, D_MODEL), jnp.float32),
        mesh=mesh,
        scratch_types=[
            pltpu.VMEM((K, IDX_MINOR), jnp.int32),
            pltpu.VMEM((K, IDX_MINOR), jnp.int32),
            pltpu.VMEM((K, IDX_MINOR, D_MODEL), jnp.float32),
            pltpu.VMEM((K, IDX_MINOR, D_MODEL), jnp.float32),
            pltpu.SemaphoreType.DMA,
            pltpu.SemaphoreType.DMA,
            pltpu.SemaphoreType.DMA,
            pltpu.SemaphoreType.DMA,
        ],
        compiler_params=pltpu.CompilerParams(use_tc_tiling_on_sc=False),
    )
    return kern(x2d, table)


def kernel(x, table):
    b = x.size
    assert b % (NUM_WORKERS * K * IDX_MINOR * 2) == 0
    chunks = b // (NUM_WORKERS * K * IDX_MINOR)
    x2d = x.reshape(b // IDX_MINOR, IDX_MINOR).astype(jnp.int32)
    out = _emb(x2d, table, chunks)
    return out.reshape(x.shape + (D_MODEL,))
